# Initial kernel scaffold; baseline (speedup 1.0000x reference)
import jax
import jax.numpy as jnp
from jax.experimental import pallas as pl


def kernel(costs):
    raise NotImplementedError("write your pallas kernel here")


# speedup vs baseline: 1.0451x; 1.0451x over previous
import functools

import jax
import jax.numpy as jnp
from jax.experimental import pallas as pl
from jax.experimental.pallas import tpu as pltpu

_INF = 1e9


def _matcher_body(cost_ref, out_ref, v_ref, uc_ref, p_ref, q_ref,
                  minv_ref, used_ref, way_ref):
    n = cost_ref.shape[1]
    lane = jax.lax.broadcasted_iota(jnp.int32, (1, n), 1)

    v_ref[...] = jnp.zeros((1, n), jnp.float32)
    uc_ref[...] = jnp.zeros((1, n), jnp.float32)
    p_ref[...] = jnp.full((1, n), -1, jnp.int32)
    q_ref[...] = jnp.full((1, n), -1, jnp.int32)

    def solve_row(i, _):
        minv_ref[...] = jnp.full((1, n), _INF, jnp.float32)
        used_ref[...] = jnp.zeros((1, n), jnp.float32)
        way_ref[...] = jnp.full((1, n), -1, jnp.int32)

        def cond(st):
            return jnp.logical_not(st[-1])

        def body(st):
            cur_u, ui, i0, j0, done = st
            row = cost_ref[0, pl.ds(i0, 1), :]
            used = used_ref[...]
            free = used == 0.0
            minv = minv_ref[...]
            cur = row - cur_u - v_ref[...]
            better = jnp.logical_and(free, cur < minv)
            minv = jnp.where(better, cur, minv)
            way_ref[...] = jnp.where(better, j0, way_ref[...])
            masked = jnp.where(free, minv, _INF)
            delta = jnp.min(masked)
            j1 = jnp.min(jnp.where(masked == delta, lane, n)).astype(jnp.int32)
            ui = ui + delta
            uc_ref[...] = uc_ref[...] + delta * used
            v_ref[...] = v_ref[...] - delta * used
            minv_ref[...] = jnp.where(free, minv - delta, minv)
            sel = lane == j1
            used_ref[...] = jnp.where(sel, 1.0, used)
            nxt = jnp.sum(jnp.where(sel, p_ref[...], 0)).astype(jnp.int32)
            done = nxt < 0
            i0 = jnp.where(done, i0, jnp.maximum(nxt, 0))
            uj1 = jnp.sum(jnp.where(sel, uc_ref[...], 0.0))
            cur_u = jnp.where(done, cur_u, uj1)
            return (cur_u, ui, i0, j1, done)

        st0 = (jnp.float32(0.0), jnp.float32(0.0), jnp.int32(i),
               jnp.int32(-1), jnp.array(False))
        _, ui, _, j1, _ = jax.lax.while_loop(cond, body, st0)

        def aug_cond(j0):
            return j0 >= 0

        def aug_body(j0):
            selj = lane == j0
            jprev = jnp.sum(jnp.where(selj, way_ref[...], 0)).astype(jnp.int32)
            selp = lane == jnp.maximum(jprev, 0)
            prow = jnp.sum(jnp.where(selp, p_ref[...], 0)).astype(jnp.int32)
            pu = jnp.sum(jnp.where(selp, uc_ref[...], 0.0))
            is_free = jprev < 0
            newrow = jnp.where(is_free, i, prow)
            newu = jnp.where(is_free, ui, pu)
            p_ref[...] = jnp.where(selj, newrow, p_ref[...])
            uc_ref[...] = jnp.where(selj, newu, uc_ref[...])
            q_ref[...] = jnp.where(lane == newrow, j0, q_ref[...])
            return jprev

        jax.lax.while_loop(aug_cond, aug_body, j1)
        return 0

    jax.lax.fori_loop(0, n, solve_row, 0)
    out_ref[0] = q_ref[...]


@functools.partial(jax.jit, static_argnames=("interpret",))
def _match(costs, interpret=False):
    bs, nq, nt = costs.shape
    return pl.pallas_call(
        _matcher_body,
        out_shape=jax.ShapeDtypeStruct((bs, 1, nq), jnp.int32),
        grid=(bs,),
        in_specs=[pl.BlockSpec((1, nq, nt), lambda b: (b, 0, 0))],
        out_specs=pl.BlockSpec((1, 1, nq), lambda b: (b, 0, 0)),
        scratch_shapes=[
            pltpu.VMEM((1, nq), jnp.float32),
            pltpu.VMEM((1, nq), jnp.float32),
            pltpu.VMEM((1, nq), jnp.int32),
            pltpu.VMEM((1, nq), jnp.int32),
            pltpu.VMEM((1, nq), jnp.float32),
            pltpu.VMEM((1, nq), jnp.float32),
            pltpu.VMEM((1, nq), jnp.int32),
        ],
        compiler_params=pltpu.CompilerParams(
            dimension_semantics=("parallel",),
        ),
        name="hungarian_jv",
        interpret=interpret,
    )(costs).reshape(bs, nq)


def kernel(costs):
    bs, nq, _ = costs.shape
    cols = _match(costs)
    rows = jnp.broadcast_to(jnp.arange(nq, dtype=jnp.int64), (bs, nq))
    return rows, cols.astype(jnp.int64)
